# Initial kernel scaffold; baseline (speedup 1.0000x reference)
import functools

import jax
import jax.numpy as jnp
from jax import lax
from jax.experimental import pallas as pl
from jax.experimental.pallas import tpu as pltpu
from jax.experimental.pallas import tpu_sc as plsc

N = 10000
D = 128
NPAD = 10112
ROWS_PER_TILE = NPAD // 16
E = 320000
NW = 32
CHUNK = 128
NCHUNK = 79
EPAD = NW * NCHUNK * CHUNK
NPAD_ROWS = NPAD - N

_mesh = plsc.VectorSubcoreMesh(core_axis_name="c", subcore_axis_name="s")


@functools.partial(
    pl.kernel,
    out_type=jax.ShapeDtypeStruct((2, NPAD), jnp.float32),
    mesh=_mesh,
    scratch_types=[
        pltpu.VMEM((NCHUNK, CHUNK), jnp.int32),
        pltpu.VMEM((CHUNK,), jnp.float32),
        pltpu.VMEM_SHARED((NPAD,), jnp.float32),
    ],
)
def _deg_kernel(dst_hbm, zeros_hbm, out_hbm, dst_v, ones_v, acc_s):
    c = lax.axis_index("c")
    s = lax.axis_index("s")
    wid = s * 2 + c
    r0 = s * ROWS_PER_TILE
    pltpu.sync_copy(dst_hbm.at[wid], dst_v)
    for i in range(CHUNK // 16):
        ones_v[pl.ds(i * 16, 16)] = jnp.full((16,), 1.0, jnp.float32)
    pltpu.sync_copy(zeros_hbm.at[pl.ds(r0, ROWS_PER_TILE)],
                    acc_s.at[pl.ds(r0, ROWS_PER_TILE)])
    plsc.subcore_barrier()

    def body(j, carry):
        pltpu.sync_copy(ones_v, acc_s.at[dst_v.at[j]], add=True)
        return carry

    lax.fori_loop(0, NCHUNK, body, 0)
    plsc.subcore_barrier()
    pltpu.sync_copy(acc_s.at[pl.ds(r0, ROWS_PER_TILE)],
                    out_hbm.at[c, pl.ds(r0, ROWS_PER_TILE)])


def _make_agg_kernel(F):
    @functools.partial(
        pl.kernel,
        out_type=jax.ShapeDtypeStruct((2, NPAD, F), jnp.float32),
        mesh=_mesh,
        scratch_types=[
            pltpu.VMEM((NCHUNK, CHUNK), jnp.int32),
            pltpu.VMEM((NCHUNK, CHUNK), jnp.int32),
            pltpu.VMEM((CHUNK, F), jnp.float32),
            pltpu.VMEM_SHARED((NPAD, F), jnp.float32),
            pltpu.VMEM_SHARED((NPAD, F), jnp.float32),
        ],
        name=f"gcn_agg_{F}",
    )
    def agg(table_hbm, src_hbm, dst_hbm, zeros_hbm, out_hbm,
            src_v, dst_v, rows_v, table_s, acc_s):
        c = lax.axis_index("c")
        s = lax.axis_index("s")
        wid = s * 2 + c
        r0 = s * ROWS_PER_TILE
        pltpu.sync_copy(src_hbm.at[wid], src_v)
        pltpu.sync_copy(dst_hbm.at[wid], dst_v)
        pltpu.sync_copy(table_hbm.at[pl.ds(r0, ROWS_PER_TILE)],
                        table_s.at[pl.ds(r0, ROWS_PER_TILE)])
        pltpu.sync_copy(zeros_hbm.at[pl.ds(r0, ROWS_PER_TILE)],
                        acc_s.at[pl.ds(r0, ROWS_PER_TILE)])
        plsc.subcore_barrier()

        def body(j, carry):
            pltpu.sync_copy(table_s.at[src_v.at[j]], rows_v)
            pltpu.sync_copy(rows_v, acc_s.at[dst_v.at[j]], add=True)
            return carry

        lax.fori_loop(0, NCHUNK, body, 0)
        plsc.subcore_barrier()
        pltpu.sync_copy(acc_s.at[pl.ds(r0, ROWS_PER_TILE)],
                        out_hbm.at[c, pl.ds(r0, ROWS_PER_TILE)])

    return agg


_agg16 = _make_agg_kernel(16)
_agg32 = _make_agg_kernel(32)


_SQRT_HALF = 0.7071067811865476


def _gelu(x):
    return 0.5 * x * (1.0 + lax.erf(x * _SQRT_HALF))


def _row_mask():
    return lax.broadcasted_iota(jnp.int32, (NPAD, 1), 0) < N


def _bn_gelu(h, g, b):
    mask = _row_mask()
    hm = jnp.where(mask, h, 0.0)
    mean = jnp.sum(hm, axis=0, keepdims=True) * (1.0 / N)
    d = h - mean
    dm = jnp.where(mask, d * d, 0.0)
    var = jnp.sum(dm, axis=0, keepdims=True) * (1.0 / N)
    hn = d * lax.rsqrt(var + 1e-5) * g + b
    return _gelu(hn)


def _d0_body(deg_ref, x_ref, w1_ref, hs1_ref, dinv_ref):
    deg = deg_ref[0] + deg_ref[1] + 1.0
    dinv = lax.rsqrt(deg)
    dinv_ref[...] = dinv
    hs1_ref[...] = jnp.dot(x_ref[...], w1_ref[...],
                           preferred_element_type=jnp.float32) * dinv


def _d1_body(acc_ref, hs1_ref, dinv_ref, b_ref, g_ref, be_ref, out_ref):
    dinv = dinv_ref[...]
    agg = acc_ref[0] + acc_ref[1] + hs1_ref[...]
    h = agg * dinv + b_ref[...]
    h = _bn_gelu(h, g_ref[...], be_ref[...])
    out_ref[...] = jnp.where(_row_mask(), h * dinv, 0.0)


def _d2_body(acc_ref, hsp_ref, dinv_ref, w_ref, b_ref, g_ref, be_ref, out_ref):
    dinv = dinv_ref[...]
    agg = (acc_ref[0] + acc_ref[1] + hsp_ref[...]) * dinv
    h = jnp.dot(agg, w_ref[...], preferred_element_type=jnp.float32) + b_ref[...]
    h = _bn_gelu(h, g_ref[...], be_ref[...])
    out_ref[...] = jnp.where(_row_mask(), h * dinv, 0.0)


def _d3_body(acc_ref, hsp_ref, dinv_ref, w_ref, b_ref, g_ref, be_ref,
             wn_ref, out_ref):
    dinv = dinv_ref[...]
    agg = (acc_ref[0] + acc_ref[1] + hsp_ref[...]) * dinv
    h = jnp.dot(agg, w_ref[...], preferred_element_type=jnp.float32) + b_ref[...]
    h = _bn_gelu(h, g_ref[...], be_ref[...])
    hs_next = jnp.dot(h, wn_ref[...], preferred_element_type=jnp.float32) * dinv
    out_ref[...] = jnp.where(_row_mask(), hs_next, 0.0)


def _d4_body(acc_ref, hs4_ref, dinv_ref, b_ref, out_ref):
    logits = (acc_ref[0] + acc_ref[1] + hs4_ref[...]) * dinv_ref[...] + b_ref[...]
    colmask = lax.broadcasted_iota(jnp.int32, (NPAD, 16), 1) < 10
    lm = jnp.where(colmask, logits, -1e30)
    mx = jnp.max(lm, axis=1, keepdims=True)
    se = jnp.sum(jnp.where(colmask, jnp.exp(lm - mx), 0.0),
                 axis=1, keepdims=True)
    out_ref[...] = logits - mx - jnp.log(se)


def _tc(body, out_shapes, *args):
    return pl.pallas_call(body, out_shape=out_shapes)(*args)


def kernel(x, edge_index, W1, b1, g1, be1, W2, b2, g2, be2, W3, b3, g3, be3,
           W4, b4):
    f32 = jnp.float32
    src = edge_index[0].astype(jnp.int32)
    dst = edge_index[1].astype(jnp.int32)
    npadedge = EPAD - E
    padidx = N + (jnp.arange(npadedge, dtype=jnp.int32) % NPAD_ROWS)
    srcp = jnp.concatenate([src, padidx]).reshape(NW, NCHUNK, CHUNK)
    dstp = jnp.concatenate([dst, padidx]).reshape(NW, NCHUNK, CHUNK)

    zeros_n = jnp.zeros((NPAD,), f32)
    zeros16 = jnp.zeros((NPAD, 16), f32)
    zeros32 = jnp.zeros((NPAD, 32), f32)
    x_pad = jnp.concatenate([x, jnp.zeros((NPAD - N, D), f32)], axis=0)
    W4p = jnp.concatenate([W4, jnp.zeros((64, 6), f32)], axis=1)
    b4p = jnp.concatenate([b4, jnp.zeros((6,), f32)])

    deg_p = _deg_kernel(dstp, zeros_n)
    deg_col = deg_p.reshape(2, NPAD, 1)

    hs1, dinv = _tc(
        _d0_body,
        [jax.ShapeDtypeStruct((NPAD, 16), f32),
         jax.ShapeDtypeStruct((NPAD, 1), f32)],
        deg_col, x_pad, W1)

    acc1 = _agg16(hs1, srcp, dstp, zeros16)
    hs2 = _tc(_d1_body, jax.ShapeDtypeStruct((NPAD, 16), f32),
              acc1, hs1, dinv, b1.reshape(1, 16), g1.reshape(1, 16),
              be1.reshape(1, 16))

    acc2 = _agg16(hs2, srcp, dstp, zeros16)
    hs3 = _tc(_d2_body, jax.ShapeDtypeStruct((NPAD, 32), f32),
              acc2, hs2, dinv, W2, b2.reshape(1, 32), g2.reshape(1, 32),
              be2.reshape(1, 32))

    acc3 = _agg32(hs3, srcp, dstp, zeros32)
    hs4 = _tc(_d3_body, jax.ShapeDtypeStruct((NPAD, 16), f32),
              acc3, hs3, dinv, W3, b3.reshape(1, 64), g3.reshape(1, 64),
              be3.reshape(1, 64), W4p)

    acc4 = _agg16(hs4, srcp, dstp, zeros16)
    out = _tc(_d4_body, jax.ShapeDtypeStruct((NPAD, 16), f32),
              acc4, hs4, dinv, b4p.reshape(1, 16))

    return out[:N, :10]


# speedup vs baseline: 27.1219x; 27.1219x over previous
import functools

import jax
import jax.numpy as jnp
from jax import lax
from jax.experimental import pallas as pl
from jax.experimental.pallas import tpu as pltpu
from jax.experimental.pallas import tpu_sc as plsc

N = 10000
D = 128
NPAD = 10240
ROWS_PER_TILE = NPAD // 16
E = 320000
NW = 32
CHUNK = 128
NCHUNK = 79
EPAD = NW * NCHUNK * CHUNK
NPAD_ROWS = NPAD - N


def _mesh():
    return plsc.VectorSubcoreMesh(core_axis_name="c", subcore_axis_name="s",
                                  num_cores=2, num_subcores=16)


@functools.cache
def _make_deg_kernel():
    @functools.partial(
        pl.kernel,
        out_type=jax.ShapeDtypeStruct((2 * NPAD,), jnp.float32),
        mesh=_mesh(),
        scratch_types=[
            pltpu.VMEM((NCHUNK, CHUNK), jnp.int32),
            pltpu.VMEM((CHUNK,), jnp.float32),
            pltpu.VMEM((ROWS_PER_TILE,), jnp.float32),
            pltpu.VMEM_SHARED((NPAD,), jnp.float32),
        ],
        name="gcn_deg",
        compiler_params=pltpu.CompilerParams(use_tc_tiling_on_sc=False),
    )
    def _deg_kernel(dst_hbm, zeros_hbm, out_hbm, dst_v, ones_v, zb_v, acc_s):
        c = lax.axis_index("c")
        s = lax.axis_index("s")
        wid = s * 2 + c
        r0 = pl.multiple_of(s * ROWS_PER_TILE, 8)
        pltpu.sync_copy(dst_hbm.at[wid], dst_v)
        for i in range(CHUNK // 16):
            ones_v[pl.ds(i * 16, 16)] = jnp.full((16,), 1.0, jnp.float32)
        pltpu.sync_copy(zeros_hbm, zb_v)
        pltpu.sync_copy(zb_v, acc_s.at[pl.ds(r0, ROWS_PER_TILE)])
        plsc.subcore_barrier()

        def body(j, carry):
            pltpu.sync_copy(ones_v, acc_s.at[dst_v.at[j]], add=True)
            return carry

        lax.fori_loop(0, NCHUNK, body, 0)
        plsc.subcore_barrier()
        oo = pl.multiple_of(c * NPAD + s * ROWS_PER_TILE, 8)
        pltpu.sync_copy(acc_s.at[pl.ds(r0, ROWS_PER_TILE)], zb_v)
        pltpu.sync_copy(zb_v, out_hbm.at[pl.ds(oo, ROWS_PER_TILE)])

    return _deg_kernel


@functools.cache
def _make_agg_kernel(F):
    @functools.partial(
        pl.kernel,
        out_type=jax.ShapeDtypeStruct((2 * NPAD, F), jnp.float32),
        mesh=_mesh(),
        scratch_types=[
            pltpu.VMEM((NCHUNK, CHUNK), jnp.int32),
            pltpu.VMEM((NCHUNK, CHUNK), jnp.int32),
            pltpu.VMEM((CHUNK, F), jnp.float32),
            pltpu.VMEM((ROWS_PER_TILE, F), jnp.float32),
            pltpu.VMEM_SHARED((NPAD, F), jnp.float32),
        ],
        name=f"gcn_agg_{F}",
        compiler_params=pltpu.CompilerParams(use_tc_tiling_on_sc=False),
    )
    def agg(table_hbm, src_hbm, dst_hbm, zeros_hbm, out_hbm,
            src_v, dst_v, rows_v, zb_v, acc_s):
        c = lax.axis_index("c")
        s = lax.axis_index("s")
        wid = s * 2 + c
        r0 = pl.multiple_of(s * ROWS_PER_TILE, 8)
        pltpu.sync_copy(src_hbm.at[wid], src_v)
        pltpu.sync_copy(dst_hbm.at[wid], dst_v)
        pltpu.sync_copy(zeros_hbm, zb_v)
        pltpu.sync_copy(zb_v, acc_s.at[pl.ds(r0, ROWS_PER_TILE)])
        plsc.subcore_barrier()

        def body(j, carry):
            pltpu.sync_copy(table_hbm.at[src_v.at[j]], rows_v)
            pltpu.sync_copy(rows_v, acc_s.at[dst_v.at[j]], add=True)
            return carry

        lax.fori_loop(0, NCHUNK, body, 0)
        plsc.subcore_barrier()
        oo = pl.multiple_of(c * NPAD + s * ROWS_PER_TILE, 8)
        pltpu.sync_copy(acc_s.at[pl.ds(r0, ROWS_PER_TILE)], zb_v)
        pltpu.sync_copy(zb_v, out_hbm.at[pl.ds(oo, ROWS_PER_TILE)])

    return agg


_SQRT_HALF = 0.7071067811865476


def _gelu(x):
    return 0.5 * x * (1.0 + lax.erf(x * _SQRT_HALF))


def _row_mask():
    return lax.broadcasted_iota(jnp.int32, (NPAD, 1), 0) < N


def _bn_gelu(h, g, b):
    mask = _row_mask()
    hm = jnp.where(mask, h, 0.0)
    mean = jnp.sum(hm, axis=0, keepdims=True) * (1.0 / N)
    d = h - mean
    dm = jnp.where(mask, d * d, 0.0)
    var = jnp.sum(dm, axis=0, keepdims=True) * (1.0 / N)
    hn = d * lax.rsqrt(var + 1e-5) * g + b
    return _gelu(hn)


def _d0_body(deg_ref, x_ref, w1_ref, hs1_ref, dinv_ref):
    deg = deg_ref[0] + deg_ref[1] + 1.0
    dinv = lax.rsqrt(deg)
    dinv_ref[...] = dinv
    hs1_ref[...] = jnp.dot(x_ref[...], w1_ref[...],
                           preferred_element_type=jnp.float32) * dinv


def _d1_body(acc_ref, hs1_ref, dinv_ref, b_ref, g_ref, be_ref, out_ref):
    dinv = dinv_ref[...]
    agg = acc_ref[0] + acc_ref[1] + hs1_ref[...]
    h = agg * dinv + b_ref[...]
    h = _bn_gelu(h, g_ref[...], be_ref[...])
    out_ref[...] = jnp.where(_row_mask(), h * dinv, 0.0)


def _d2_body(acc_ref, hsp_ref, dinv_ref, w_ref, b_ref, g_ref, be_ref, out_ref):
    dinv = dinv_ref[...]
    agg = (acc_ref[0] + acc_ref[1] + hsp_ref[...]) * dinv
    h = jnp.dot(agg, w_ref[...], preferred_element_type=jnp.float32) + b_ref[...]
    h = _bn_gelu(h, g_ref[...], be_ref[...])
    out_ref[...] = jnp.where(_row_mask(), h * dinv, 0.0)


def _d3_body(acc_ref, hsp_ref, dinv_ref, w_ref, b_ref, g_ref, be_ref,
             wn_ref, out_ref):
    dinv = dinv_ref[...]
    agg = (acc_ref[0] + acc_ref[1] + hsp_ref[...]) * dinv
    h = jnp.dot(agg, w_ref[...], preferred_element_type=jnp.float32) + b_ref[...]
    h = _bn_gelu(h, g_ref[...], be_ref[...])
    hs_next = jnp.dot(h, wn_ref[...], preferred_element_type=jnp.float32) * dinv
    out_ref[...] = jnp.where(_row_mask(), hs_next, 0.0)


def _d4_body(acc_ref, hs4_ref, dinv_ref, b_ref, out_ref):
    logits = (acc_ref[0] + acc_ref[1] + hs4_ref[...]) * dinv_ref[...] + b_ref[...]
    colmask = lax.broadcasted_iota(jnp.int32, (NPAD, 16), 1) < 10
    lm = jnp.where(colmask, logits, -1e30)
    mx = jnp.max(lm, axis=1, keepdims=True)
    se = jnp.sum(jnp.where(colmask, jnp.exp(lm - mx), 0.0),
                 axis=1, keepdims=True)
    out_ref[...] = logits - mx - jnp.log(se)


def _tc(body, out_shapes, *args):
    return pl.pallas_call(body, out_shape=out_shapes)(*args)


def kernel(x, edge_index, W1, b1, g1, be1, W2, b2, g2, be2, W3, b3, g3, be3,
           W4, b4):
    f32 = jnp.float32
    src = edge_index[0].astype(jnp.int32)
    dst = edge_index[1].astype(jnp.int32)
    npadedge = EPAD - E
    padidx = N + (jnp.arange(npadedge, dtype=jnp.int32) % NPAD_ROWS)
    srcp = jnp.concatenate([src, padidx]).reshape(NW, NCHUNK, CHUNK)
    dstp = jnp.concatenate([dst, padidx]).reshape(NW, NCHUNK, CHUNK)

    zeros_n = jnp.zeros((ROWS_PER_TILE,), f32)
    zeros16 = jnp.zeros((ROWS_PER_TILE, 16), f32)
    zeros32 = jnp.zeros((ROWS_PER_TILE, 32), f32)
    x_pad = jnp.concatenate([x, jnp.zeros((NPAD - N, D), f32)], axis=0)
    W4p = jnp.concatenate([W4, jnp.zeros((64, 6), f32)], axis=1)
    b4p = jnp.concatenate([b4, jnp.zeros((6,), f32)])

    _deg = _make_deg_kernel()
    _agg16 = _make_agg_kernel(16)
    _agg32 = _make_agg_kernel(32)

    deg_p = _deg(dstp, zeros_n)
    deg_col = deg_p.reshape(2, NPAD, 1)

    hs1, dinv = _tc(
        _d0_body,
        [jax.ShapeDtypeStruct((NPAD, 16), f32),
         jax.ShapeDtypeStruct((NPAD, 1), f32)],
        deg_col, x_pad, W1)

    acc1 = _agg16(hs1, srcp, dstp, zeros16).reshape(2, NPAD, 16)
    hs2 = _tc(_d1_body, jax.ShapeDtypeStruct((NPAD, 16), f32),
              acc1, hs1, dinv, b1.reshape(1, 16), g1.reshape(1, 16),
              be1.reshape(1, 16))

    acc2 = _agg16(hs2, srcp, dstp, zeros16).reshape(2, NPAD, 16)
    hs3 = _tc(_d2_body, jax.ShapeDtypeStruct((NPAD, 32), f32),
              acc2, hs2, dinv, W2, b2.reshape(1, 32), g2.reshape(1, 32),
              be2.reshape(1, 32))

    acc3 = _agg32(hs3, srcp, dstp, zeros32).reshape(2, NPAD, 32)
    hs4 = _tc(_d3_body, jax.ShapeDtypeStruct((NPAD, 16), f32),
              acc3, hs3, dinv, W3, b3.reshape(1, 64), g3.reshape(1, 64),
              be3.reshape(1, 64), W4p)

    acc4 = _agg16(hs4, srcp, dstp, zeros16).reshape(2, NPAD, 16)
    out = _tc(_d4_body, jax.ShapeDtypeStruct((NPAD, 16), f32),
              acc4, hs4, dinv, b4p.reshape(1, 16))

    return out[:N, :10]
